# Initial kernel scaffold; baseline (speedup 1.0000x reference)
import functools

import jax
import jax.numpy as jnp
from jax import lax
from jax.experimental import pallas as pl
from jax.experimental.pallas import tpu as pltpu
from jax.experimental.pallas import tpu_sc as plsc

NUM_CLS = 7
BATCH = 16384
SMOOTHING = 0.1
CONFIDENCE = 1.0 - SMOOTHING
SMOOTH_VAL = SMOOTHING / (NUM_CLS - 1)

NUM_WORKERS = 32
B_PER_W = BATCH // NUM_WORKERS
W_PER_W = B_PER_W * NUM_CLS
LANES = 16
GROUPS = B_PER_W // LANES

_mesh = plsc.VectorSubcoreMesh(core_axis_name="c", subcore_axis_name="s")


@functools.partial(
    pl.kernel,
    out_type=jax.ShapeDtypeStruct((BATCH * NUM_CLS,), jnp.float32),
    mesh=_mesh,
    scratch_types=[
        pltpu.VMEM((B_PER_W,), jnp.int32),
        pltpu.VMEM((W_PER_W,), jnp.float32),
    ],
)
def _smooth_kernel(tgt_hbm, out_hbm, tgt_v, out_v):
    wid = lax.axis_index("s") * 2 + lax.axis_index("c")
    pltpu.sync_copy(tgt_hbm.at[pl.ds(wid * B_PER_W, B_PER_W)], tgt_v)

    smooth_v = jnp.full((LANES,), SMOOTH_VAL, jnp.float32)
    conf_v = jnp.full((LANES,), CONFIDENCE, jnp.float32)
    row_off = lax.iota(jnp.int32, LANES) * NUM_CLS

    def body(j, _):
        off = j * (LANES * NUM_CLS)
        for k in range(NUM_CLS):
            out_v[pl.ds(off + k * LANES, LANES)] = smooth_v
        tgt = tgt_v[pl.ds(j * LANES, LANES)]
        plsc.store_scatter(out_v, [off + row_off + tgt], conf_v)
        return 0

    lax.fori_loop(0, GROUPS, body, 0, unroll=4)
    pltpu.sync_copy(out_v, out_hbm.at[pl.ds(wid * W_PER_W, W_PER_W)])


def kernel(targets):
    out = _smooth_kernel(targets.astype(jnp.int32))
    return out.reshape(BATCH, NUM_CLS)


# speedup vs baseline: 1.7147x; 1.7147x over previous
import functools

import jax
import jax.numpy as jnp
from jax import lax
from jax.experimental import pallas as pl
from jax.experimental.pallas import tpu as pltpu
from jax.experimental.pallas import tpu_sc as plsc

NUM_CLS = 7
BATCH = 16384
SMOOTHING = 0.1
CONFIDENCE = 1.0 - SMOOTHING
SMOOTH_VAL = SMOOTHING / (NUM_CLS - 1)

NUM_WORKERS = 32
B_PER_W = BATCH // NUM_WORKERS
W_PER_W = B_PER_W * NUM_CLS
LANES = 16
GROUPS = B_PER_W // LANES

_mesh = plsc.VectorSubcoreMesh(core_axis_name="c", subcore_axis_name="s")


def _smooth_body(tgt_hbm, out_hbm, tgt_v, out_v):
    wid = lax.axis_index("s") * 2 + lax.axis_index("c")
    pltpu.sync_copy(tgt_hbm.at[pl.ds(wid * B_PER_W, B_PER_W)], tgt_v)

    smooth_v = jnp.full((LANES,), SMOOTH_VAL, jnp.float32)
    conf_v = jnp.full((LANES,), CONFIDENCE, jnp.float32)
    row_off = lax.iota(jnp.int32, LANES) * NUM_CLS

    def body(j, _):
        off = j * (LANES * NUM_CLS)
        for k in range(NUM_CLS):
            out_v[pl.ds(off + k * LANES, LANES)] = smooth_v
        tgt = tgt_v[pl.ds(j * LANES, LANES)]
        plsc.store_scatter(out_v, [off + row_off + tgt], conf_v)
        return 0

    lax.fori_loop(0, GROUPS, body, 0, unroll=4)
    pltpu.sync_copy(out_v, out_hbm.at[pl.ds(wid * W_PER_W, W_PER_W)])


_smooth_kernel = functools.partial(
    pl.kernel,
    out_type=jax.ShapeDtypeStruct((BATCH * NUM_CLS,), jnp.float32),
    mesh=_mesh,
    scratch_types=[
        pltpu.VMEM((B_PER_W,), jnp.int32),
        pltpu.VMEM((W_PER_W,), jnp.float32),
    ],
    compiler_params=pltpu.CompilerParams(needs_layout_passes=False),
)(_smooth_body)


def kernel(targets):
    out = _smooth_kernel(targets.astype(jnp.int32))
    return out.reshape(BATCH, NUM_CLS)
